# baseline (speedup 1.0000x reference)
import jax
import jax.numpy as jnp
from jax.experimental import pallas as pl


def _gather_kernel(x_ref, o_ref):
    o_ref[...] = jnp.concatenate(
        [x_ref[0:1, :], x_ref[2:3, :], x_ref[1:2, :], x_ref[3:4, :]], axis=0
    )


def kernel(x):
    out = pl.pallas_call(
        _gather_kernel,
        out_shape=jax.ShapeDtypeStruct((4, 64), jnp.float32),
        grid=(1,),
        in_specs=[pl.BlockSpec((8, 64), lambda i: (0, 0))],
        out_specs=pl.BlockSpec((4, 64), lambda i: (0, 0)),
    )(x)
    return out.reshape(1, 2, 2, 64)


# speedup vs baseline: 231.6272x; 231.6272x over previous
import jax
import jax.numpy as jnp
from jax.experimental import pallas as pl


def _gather_kernel(xt_ref, o_ref):
    t = jnp.transpose(xt_ref[...])
    o_ref[...] = jnp.concatenate(
        [t[0:1, :], t[2:3, :], t[1:2, :], t[3:4, :]], axis=0
    )


def kernel(x):
    xt = x.T
    out = pl.pallas_call(
        _gather_kernel,
        out_shape=jax.ShapeDtypeStruct((4, 64), jnp.float32),
        grid=(1,),
        in_specs=[pl.BlockSpec((64, 128), lambda i: (0, 0))],
        out_specs=pl.BlockSpec((4, 64), lambda i: (0, 0)),
    )(xt)
    return out.reshape(1, 2, 2, 64)
